# baseline (speedup 1.0000x reference)
import functools

import jax
import jax.numpy as jnp
from jax import lax
from jax.experimental import pallas as pl
from jax.experimental.pallas import tpu as pltpu
from jax.experimental.pallas import tpu_sc as plsc

_NC = 2
_NS = 16
_NW = _NC * _NS
_L = 16
_R = 2000


def _repack_body(t_ref, out_ref):
    r = out_ref.shape[0]
    d = t_ref.shape[1]
    for k in range(4):
        out_ref[:, d * k:d * (k + 1)] = t_ref[k * r:(k + 1) * r, :]


def _tc_repack(table):
    V, D = table.shape
    nblk = V // (4 * _R)
    return pl.pallas_call(
        _repack_body,
        grid=(nblk,),
        in_specs=[pl.BlockSpec((4 * _R, D), lambda i: (i, 0))],
        out_specs=pl.BlockSpec((_R, 4 * D), lambda i: (i, 0)),
        out_shape=jax.ShapeDtypeStruct((V // 4, 4 * D), table.dtype),
    )(table)


def _sc_gather4(table4, idx):
    B, = idx.shape
    D4 = table4.shape[1]
    b_per_w = B // _NW
    G = 4 * _R

    @functools.partial(
        pl.kernel,
        mesh=plsc.VectorSubcoreMesh(core_axis_name="c", subcore_axis_name="s"),
        out_type=jax.ShapeDtypeStruct((B, D4), table4.dtype),
        scratch_types=[
            pltpu.VMEM((b_per_w,), jnp.int32),
            pltpu.VMEM((b_per_w,), jnp.int32),
            pltpu.VMEM((b_per_w, D4), table4.dtype),
            pltpu.SemaphoreType.DMA,
        ],
        compiler_params=pltpu.CompilerParams(use_tc_tiling_on_sc=True),
    )
    def gather_k(table_hbm, idx_hbm, out_hbm, idx_v, idx2_v, rows_v, sem):
        wid = lax.axis_index("s") * _NC + lax.axis_index("c")
        base = wid * b_per_w
        pltpu.sync_copy(idx_hbm.at[pl.ds(base, b_per_w)], idx_v)
        for g in range(b_per_w // _L):
            v = idx_v[pl.ds(g * _L, _L)]
            p = lax.rem(v, G)
            idx2_v[pl.ds(g * _L, _L)] = lax.div(v, G) * _R + lax.rem(p, _R)
        pltpu.async_copy(table_hbm.at[idx2_v], rows_v, sem).wait()
        pltpu.sync_copy(rows_v, out_hbm.at[pl.ds(base, b_per_w)])

    return gather_k(table4, idx)


def _tc_score_body(theta4_ref, uidx_ref, alpha_ref, out_ref):
    u = uidx_ref[...]
    sub = lax.rem(u, 4 * _R) // _R
    t4 = theta4_ref[...]
    D = t4.shape[1] // 4
    theta = jnp.where(sub == 0, t4[:, 0:D], t4[:, D:2 * D])
    theta = jnp.where(sub == 2, t4[:, 2 * D:3 * D], theta)
    theta = jnp.where(sub == 3, t4[:, 3 * D:4 * D], theta)
    util = jnp.dot(theta, alpha_ref[...], preferred_element_type=jnp.float32)
    m = jnp.max(util, axis=-1, keepdims=True)
    s = util - m
    lse = jnp.log(jnp.sum(jnp.exp(s), axis=-1, keepdims=True))
    out_ref[...] = s - lse


def _tc_score(theta4, uidx, alpha_t, block_b=1024):
    B, D4 = theta4.shape
    N = alpha_t.shape[1]
    return pl.pallas_call(
        _tc_score_body,
        grid=(B // block_b,),
        in_specs=[
            pl.BlockSpec((block_b, D4), lambda i: (i, 0)),
            pl.BlockSpec((block_b, 1), lambda i: (i, 0)),
            pl.BlockSpec((alpha_t.shape[0], N), lambda i: (0, 0)),
        ],
        out_specs=pl.BlockSpec((block_b, N), lambda i: (i, 0)),
        out_shape=jax.ShapeDtypeStruct((B, N), jnp.float32),
    )(theta4, uidx, alpha_t)


def kernel(user_index, theta_mean, alpha_mean):
    idx = user_index.astype(jnp.int32)
    table4 = _tc_repack(theta_mean)
    theta4 = _sc_gather4(table4, idx)
    alpha_t = alpha_mean.T
    return _tc_score(theta4, idx.reshape(-1, 1), alpha_t)


# speedup vs baseline: 1.6329x; 1.6329x over previous
import functools

import jax
import jax.numpy as jnp
from jax import lax
from jax.experimental import pallas as pl
from jax.experimental.pallas import tpu as pltpu
from jax.experimental.pallas import tpu_sc as plsc

_NC = 2
_NS = 16
_NW = _NC * _NS


def _sc_gather(table, idx):
    B, = idx.shape
    D = table.shape[1]
    b_per_w = B // _NW

    @functools.partial(
        pl.kernel,
        mesh=plsc.VectorSubcoreMesh(core_axis_name="c", subcore_axis_name="s"),
        out_type=jax.ShapeDtypeStruct((B, D), table.dtype),
        scratch_types=[
            pltpu.VMEM((b_per_w,), jnp.int32),
            pltpu.VMEM((b_per_w, D), table.dtype),
            pltpu.SemaphoreType.DMA,
        ],
        compiler_params=pltpu.CompilerParams(use_tc_tiling_on_sc=True),
    )
    def gather_k(table_hbm, idx_hbm, out_hbm, idx_v, rows_v, sem):
        wid = lax.axis_index("s") * _NC + lax.axis_index("c")
        base = wid * b_per_w
        pltpu.sync_copy(idx_hbm.at[pl.ds(base, b_per_w)], idx_v)

        def body(g, carry):
            v = idx_v[pl.ds(g * 16, 16)]
            for j in range(16):
                pltpu.async_copy(table_hbm.at[pl.ds(v[j], 1)],
                                 rows_v.at[pl.ds(g * 16 + j, 1)], sem)
            return carry

        lax.fori_loop(0, b_per_w // 16, body, 0)
        pltpu.make_async_copy(table_hbm.at[pl.ds(0, b_per_w)],
                              rows_v, sem).wait()
        pltpu.sync_copy(rows_v, out_hbm.at[pl.ds(base, b_per_w)])

    return gather_k(table, idx)


def _tc_score_body(theta_ref, alpha_ref, out_ref):
    util = jnp.dot(theta_ref[...], alpha_ref[...],
                   preferred_element_type=jnp.float32)
    m = jnp.max(util, axis=-1, keepdims=True)
    s = util - m
    lse = jnp.log(jnp.sum(jnp.exp(s), axis=-1, keepdims=True))
    out_ref[...] = s - lse


def _tc_score(theta, alpha_t, block_b=1024):
    B, D = theta.shape
    N = alpha_t.shape[1]
    return pl.pallas_call(
        _tc_score_body,
        grid=(B // block_b,),
        in_specs=[
            pl.BlockSpec((block_b, D), lambda i: (i, 0)),
            pl.BlockSpec((D, N), lambda i: (0, 0)),
        ],
        out_specs=pl.BlockSpec((block_b, N), lambda i: (i, 0)),
        out_shape=jax.ShapeDtypeStruct((B, N), jnp.float32),
    )(theta, alpha_t)


def kernel(user_index, theta_mean, alpha_mean):
    idx = user_index.astype(jnp.int32)
    theta = _sc_gather(theta_mean, idx)
    alpha_t = alpha_mean.T
    return _tc_score(theta, alpha_t)


# speedup vs baseline: 1.6455x; 1.0077x over previous
import functools

import jax
import jax.numpy as jnp
from jax import lax
from jax.experimental import pallas as pl
from jax.experimental.pallas import tpu as pltpu
from jax.experimental.pallas import tpu_sc as plsc

_NC = 2
_NS = 16
_NW = _NC * _NS


def _sc_gather(table, idx):
    B, = idx.shape
    D = table.shape[1]
    b_per_w = B // _NW

    @functools.partial(
        pl.kernel,
        mesh=plsc.VectorSubcoreMesh(core_axis_name="c", subcore_axis_name="s"),
        out_type=jax.ShapeDtypeStruct((B, D), table.dtype),
        scratch_types=[
            pltpu.VMEM((b_per_w,), jnp.int32),
            pltpu.VMEM((b_per_w, D), table.dtype),
            pltpu.SemaphoreType.DMA,
        ],
        compiler_params=pltpu.CompilerParams(use_tc_tiling_on_sc=True),
    )
    def gather_k(table_hbm, idx_hbm, out_hbm, idx_v, rows_v, sem):
        wid = lax.axis_index("s") * _NC + lax.axis_index("c")
        base = wid * b_per_w
        pltpu.sync_copy(idx_hbm.at[pl.ds(base, b_per_w)], idx_v)

        def body(g, carry):
            v = idx_v[pl.ds(g * 16, 16)]
            for j in range(16):
                pltpu.async_copy(table_hbm.at[pl.ds(v[j], 1)],
                                 rows_v.at[pl.ds(g * 16 + j, 1)], sem)
            return carry

        lax.fori_loop(0, b_per_w // 16, body, 0)
        pltpu.make_async_copy(table_hbm.at[pl.ds(0, b_per_w)],
                              rows_v, sem).wait()
        pltpu.sync_copy(rows_v, out_hbm.at[pl.ds(base, b_per_w)])

    return gather_k(table, idx)


def _tc_score_body(theta_ref, alpha_ref, out_ref):
    util = jnp.dot(theta_ref[...], alpha_ref[...],
                   preferred_element_type=jnp.float32)
    m = jnp.max(util, axis=-1, keepdims=True)
    s = util - m
    lse = jnp.log(jnp.sum(jnp.exp(s), axis=-1, keepdims=True))
    out_ref[...] = s - lse


def _tc_score(theta, alpha_t, block_b=2048):
    B, D = theta.shape
    N = alpha_t.shape[1]
    return pl.pallas_call(
        _tc_score_body,
        grid=(B // block_b,),
        in_specs=[
            pl.BlockSpec((block_b, D), lambda i: (i, 0)),
            pl.BlockSpec((D, N), lambda i: (0, 0)),
        ],
        out_specs=pl.BlockSpec((block_b, N), lambda i: (i, 0)),
        out_shape=jax.ShapeDtypeStruct((B, N), jnp.float32),
    )(theta, alpha_t)


def kernel(user_index, theta_mean, alpha_mean):
    idx = user_index.astype(jnp.int32)
    theta = _sc_gather(theta_mean, idx)
    alpha_t = alpha_mean.T
    return _tc_score(theta, alpha_t)
